# baseline (speedup 1.0000x reference)
import functools

import jax
import jax.numpy as jnp
from jax import lax
from jax.experimental import pallas as pl
from jax.experimental.pallas import tpu as pltpu
from jax.experimental.pallas import tpu_sc as plsc

NC = 2
NS = 16
NW = NC * NS
LANES = 16

CHUNK_A = 104
CHUNK_B = 96


@functools.partial(jax.jit, static_argnames=())
def kernel(x, token_table, pos_table):
    B, T = x.shape
    V, D = token_table.shape
    rows_per_w = B // NW

    mesh = plsc.VectorSubcoreMesh(core_axis_name="c", subcore_axis_name="s")

    @functools.partial(
        pl.kernel,
        mesh=mesh,
        compiler_params=pltpu.CompilerParams(use_tc_tiling_on_sc=False),
        out_type=jax.ShapeDtypeStruct((B * T, D), jnp.float32),
        scratch_types=[
            pltpu.VMEM((T,), jnp.int32),
            pltpu.VMEM((T, D), jnp.float32),
            pltpu.VMEM((T, D), jnp.float32),
            pltpu.SemaphoreType.DMA,
            pltpu.SemaphoreType.DMA,
        ],
    )
    def emb(x_hbm, tok_hbm, pos_hbm, out_hbm, idx_v, rows_v, pos_v, sem_a, sem_b):
        wid = lax.axis_index("s") * NC + lax.axis_index("c")
        base_row = wid * rows_per_w

        pltpu.sync_copy(pos_hbm.at[pl.ds(0, T)], pos_v)

        @pl.loop(0, rows_per_w)
        def _(r):
            flat = (base_row + r) * T
            pltpu.sync_copy(x_hbm.at[pl.ds(flat, T)], idx_v)
            cp_a = pltpu.async_copy(
                tok_hbm.at[idx_v.at[pl.ds(0, CHUNK_A)]],
                rows_v.at[pl.ds(0, CHUNK_A)],
                sem_a,
            )
            cp_b = pltpu.async_copy(
                tok_hbm.at[idx_v.at[pl.ds(CHUNK_A, CHUNK_B)]],
                rows_v.at[pl.ds(CHUNK_A, CHUNK_B)],
                sem_b,
            )
            cp_a.wait()
            cp_b.wait()

            @pl.loop(0, T)
            def _(i):
                for c in range(D // LANES):
                    sl = (i, pl.ds(c * LANES, LANES))
                    rows_v[sl] = rows_v[sl] + pos_v[sl]

            pltpu.sync_copy(rows_v, out_hbm.at[pl.ds(flat, T)])

    out = emb(x.reshape(-1), token_table, pos_table)
    return out.reshape(B, T, D)


# speedup vs baseline: 1.2015x; 1.2015x over previous
import functools

import jax
import jax.numpy as jnp
from jax import lax
from jax.experimental import pallas as pl
from jax.experimental.pallas import tpu as pltpu
from jax.experimental.pallas import tpu_sc as plsc

NC = 2
NS = 16
NW = NC * NS
LANES = 16

CHUNK_A = 104
CHUNK_B = 96
NBUF = 4


@jax.jit
def kernel(x, token_table, pos_table):
    B, T = x.shape
    V, D = token_table.shape
    R = B // NW

    mesh = plsc.VectorSubcoreMesh(core_axis_name="c", subcore_axis_name="s")

    @functools.partial(
        pl.kernel,
        mesh=mesh,
        compiler_params=pltpu.CompilerParams(use_tc_tiling_on_sc=False),
        out_type=jax.ShapeDtypeStruct((B * T, D), jnp.float32),
        scratch_types=[
            pltpu.VMEM((R * T,), jnp.int32),
            pltpu.VMEM((T, D), jnp.float32),
            [pltpu.VMEM((T, D), jnp.float32)] * NBUF,
            [pltpu.SemaphoreType.DMA] * NBUF,
            [pltpu.SemaphoreType.DMA] * NBUF,
            pltpu.SemaphoreType.DMA,
        ],
    )
    def emb(x_hbm, tok_hbm, pos_hbm, out_hbm, idx_v, pos_v, bufs, gsems, osems,
            sem0):
        wid = lax.axis_index("s") * NC + lax.axis_index("c")
        base = wid * R

        cp_pos = pltpu.async_copy(pos_hbm.at[pl.ds(0, T)], pos_v, sem0)
        cp_idx = pltpu.async_copy(x_hbm.at[pl.ds(base * T, R * T)], idx_v, sem0)
        cp_pos.wait()
        cp_idx.wait()

        def issue_gather(r, b):
            off = r * T
            pltpu.async_copy(
                tok_hbm.at[idx_v.at[pl.ds(off, CHUNK_A)]],
                bufs[b].at[pl.ds(0, CHUNK_A)],
                gsems[b],
            )
            pltpu.async_copy(
                tok_hbm.at[idx_v.at[pl.ds(off + CHUNK_A, CHUNK_B)]],
                bufs[b].at[pl.ds(CHUNK_A, CHUNK_B)],
                gsems[b],
            )

        def wait_gather(r, b):
            off = r * T
            pltpu.make_async_copy(
                tok_hbm.at[idx_v.at[pl.ds(off, CHUNK_A)]],
                bufs[b].at[pl.ds(0, CHUNK_A)],
                gsems[b],
            ).wait()
            pltpu.make_async_copy(
                tok_hbm.at[idx_v.at[pl.ds(off + CHUNK_A, CHUNK_B)]],
                bufs[b].at[pl.ds(CHUNK_A, CHUNK_B)],
                gsems[b],
            ).wait()

        issue_gather(0, 0)
        issue_gather(1, 1)

        @pl.loop(0, R, step=NBUF)
        def _(r0):
            for j in range(NBUF):
                r = r0 + j
                b = j
                b2 = (j + 2) % NBUF

                wait_gather(r, b)

                @pl.when(r + 2 < R)
                def _():
                    @pl.when(r >= 2)
                    def _():
                        pltpu.make_async_copy(
                            bufs[b2],
                            out_hbm.at[pl.ds((base + r - 2) * T, T)],
                            osems[b2],
                        ).wait()

                    issue_gather(r + 2, b2)

                @pl.loop(0, T)
                def _(t):
                    for c in range(D // LANES):
                        sl = (t, pl.ds(c * LANES, LANES))
                        plsc.addupdate(bufs[b].at[sl], pos_v[sl])

                pltpu.async_copy(
                    bufs[b],
                    out_hbm.at[pl.ds((base + r) * T, T)],
                    osems[b],
                )

        for j in range(NBUF):
            r = R - NBUF + j
            pltpu.make_async_copy(
                bufs[j],
                out_hbm.at[pl.ds((base + r) * T, T)],
                osems[j],
            ).wait()

    out = emb(x.reshape(-1), token_table, pos_table)
    return out.reshape(B, T, D)


# speedup vs baseline: 1.2020x; 1.0004x over previous
import functools

import jax
import jax.numpy as jnp
from jax import lax
from jax.experimental import pallas as pl
from jax.experimental.pallas import tpu as pltpu
from jax.experimental.pallas import tpu_sc as plsc

NC = 2
NS = 16
NW = NC * NS
LANES = 16

CHUNK_A = 104
CHUNK_B = 96
NBUF = 4


@jax.jit
def kernel(x, token_table, pos_table):
    B, T = x.shape
    V, D = token_table.shape
    R = B // NW

    mesh = plsc.VectorSubcoreMesh(core_axis_name="c", subcore_axis_name="s")

    @functools.partial(
        pl.kernel,
        mesh=mesh,
        compiler_params=pltpu.CompilerParams(use_tc_tiling_on_sc=False),
        out_type=jax.ShapeDtypeStruct((B, T, D), jnp.float32),
        scratch_types=[
            pltpu.VMEM((R, T), jnp.int32),
            pltpu.VMEM((T, D), jnp.float32),
            [pltpu.VMEM((T, D), jnp.float32)] * NBUF,
            [pltpu.SemaphoreType.DMA] * NBUF,
            [pltpu.SemaphoreType.DMA] * NBUF,
            pltpu.SemaphoreType.DMA,
        ],
    )
    def emb(x_hbm, tok_hbm, pos_hbm, out_hbm, idx_v, pos_v, bufs, gsems, osems,
            sem0):
        wid = lax.axis_index("s") * NC + lax.axis_index("c")
        base = wid * R

        cp_pos = pltpu.async_copy(pos_hbm.at[pl.ds(0, T)], pos_v, sem0)
        cp_idx = pltpu.async_copy(x_hbm.at[pl.ds(base, R)], idx_v, sem0)
        cp_pos.wait()
        cp_idx.wait()

        def issue_gather(r, b):
            pltpu.async_copy(
                tok_hbm.at[idx_v.at[r, pl.ds(0, CHUNK_A)]],
                bufs[b].at[pl.ds(0, CHUNK_A)],
                gsems[b],
            )
            pltpu.async_copy(
                tok_hbm.at[idx_v.at[r, pl.ds(CHUNK_A, CHUNK_B)]],
                bufs[b].at[pl.ds(CHUNK_A, CHUNK_B)],
                gsems[b],
            )

        def wait_gather(r, b):
            pltpu.make_async_copy(
                tok_hbm.at[idx_v.at[r, pl.ds(0, CHUNK_A)]],
                bufs[b].at[pl.ds(0, CHUNK_A)],
                gsems[b],
            ).wait()
            pltpu.make_async_copy(
                tok_hbm.at[idx_v.at[r, pl.ds(CHUNK_A, CHUNK_B)]],
                bufs[b].at[pl.ds(CHUNK_A, CHUNK_B)],
                gsems[b],
            ).wait()

        issue_gather(0, 0)
        issue_gather(1, 1)

        @pl.loop(0, R, step=NBUF)
        def _(r0):
            for j in range(NBUF):
                r = r0 + j
                b = j
                b2 = (j + 2) % NBUF

                wait_gather(r, b)

                @pl.when(r + 2 < R)
                def _():
                    @pl.when(r >= 2)
                    def _():
                        pltpu.make_async_copy(
                            bufs[b2],
                            out_hbm.at[base + r - 2],
                            osems[b2],
                        ).wait()

                    issue_gather(r + 2, b2)

                @pl.loop(0, T)
                def _(t):
                    for c in range(D // LANES):
                        sl = (t, pl.ds(c * LANES, LANES))
                        plsc.addupdate(bufs[b].at[sl], pos_v[sl])

                pltpu.async_copy(
                    bufs[b],
                    out_hbm.at[base + r],
                    osems[b],
                )

        for j in range(NBUF):
            r = R - NBUF + j
            pltpu.make_async_copy(
                bufs[j],
                out_hbm.at[base + r],
                osems[j],
            ).wait()

    return emb(x, token_table, pos_table)
